# Initial kernel scaffold; baseline (speedup 1.0000x reference)
import jax
import jax.numpy as jnp
from jax.experimental import pallas as pl


def kernel(context_tokens, time_tokens, loc_tokens, act_tokens, time_table, loc_table, act_table, age_table, gender_table, W_time, b_time, W_loc, b_loc, W_act, b_act):
    raise NotImplementedError("write your pallas kernel here")


# speedup vs baseline: 4.2816x; 4.2816x over previous
import functools

import jax
import jax.numpy as jnp
from jax import lax
from jax.experimental import pallas as pl
from jax.experimental.pallas import tpu as pltpu, tpu_sc as plsc

B = 16384
L = 200
BL = B * L

NC = 2
NS = 16
NW = NC * NS
LANES = 16

TPW = BL // NW
CHUNK = 1024
NCHUNK = TPW // CHUNK
CPW = B // NW
CTX_PAD = 16


def _proj_body(x_ref, w_ref, b_ref, o_ref):
    o_ref[...] = (
        jnp.dot(x_ref[...], w_ref[...], preferred_element_type=jnp.float32)
        + b_ref[...]
    )


def _project(x, w_t, b_row, blk):
    v, d_in = x.shape
    d_out = w_t.shape[1]
    return pl.pallas_call(
        _proj_body,
        grid=(v // blk,),
        in_specs=[
            pl.BlockSpec((blk, d_in), lambda i: (i, 0)),
            pl.BlockSpec((d_in, d_out), lambda i: (0, 0)),
            pl.BlockSpec((1, d_out), lambda i: (0, 0)),
        ],
        out_specs=pl.BlockSpec((blk, d_out), lambda i: (i, 0)),
        out_shape=jax.ShapeDtypeStruct((v, d_out), jnp.float32),
    )(x, w_t, b_row)


_MESH = plsc.VectorSubcoreMesh(core_axis_name="c", subcore_axis_name="s")
_SC_PARAMS = pltpu.CompilerParams(use_tc_tiling_on_sc=False)


@functools.partial(
    pl.kernel,
    mesh=_MESH,
    compiler_params=_SC_PARAMS,
    out_type=[
        jax.ShapeDtypeStruct((BL, 32), jnp.float32),
        jax.ShapeDtypeStruct((BL, 32), jnp.float32),
        jax.ShapeDtypeStruct((B, CTX_PAD), jnp.float32),
    ],
    scratch_types=[
        pltpu.VMEM((CHUNK,), jnp.int32),
        pltpu.VMEM((CHUNK, 32), jnp.float32),
        pltpu.VMEM((CPW,), jnp.int32),
        pltpu.VMEM((CPW,), jnp.int32),
        pltpu.VMEM((CPW,), jnp.int32),
        pltpu.VMEM((CPW, CTX_PAD), jnp.float32),
        pltpu.SemaphoreType.DMA,
    ],
)
def _gather_time_act_ctx(tt_hbm, at_hbm, c0_hbm, c1_hbm,
                         p_time_hbm, p_act_hbm, p_ctx_hbm,
                         out_t_hbm, out_a_hbm, out_c_hbm,
                         idx_v, rows_v, c0_v, c1_v, cidx_v, crows_v, sem):
    wid = lax.axis_index("s") * NC + lax.axis_index("c")

    cbase = wid * CPW
    pltpu.sync_copy(c0_hbm.at[pl.ds(cbase, CPW)], c0_v)
    pltpu.sync_copy(c1_hbm.at[pl.ds(cbase, CPW)], c1_v)
    for j in range(CPW // LANES):
        sl = pl.ds(j * LANES, LANES)
        cidx_v[sl] = c0_v[sl] * 100 + c1_v[sl]
    pltpu.async_copy(p_ctx_hbm.at[cidx_v], crows_v, sem).wait()
    pltpu.sync_copy(crows_v, out_c_hbm.at[pl.ds(cbase, CPW)])

    def chunk(i, carry):
        off = wid * TPW + i * CHUNK
        pltpu.sync_copy(tt_hbm.at[pl.ds(off, CHUNK)], idx_v)
        pltpu.async_copy(p_time_hbm.at[idx_v], rows_v, sem).wait()
        pltpu.sync_copy(rows_v, out_t_hbm.at[pl.ds(off, CHUNK)])
        pltpu.sync_copy(at_hbm.at[pl.ds(off, CHUNK)], idx_v)
        pltpu.async_copy(p_act_hbm.at[idx_v], rows_v, sem).wait()
        pltpu.sync_copy(rows_v, out_a_hbm.at[pl.ds(off, CHUNK)])
        return carry

    lax.fori_loop(0, NCHUNK, chunk, 0)


@functools.partial(
    pl.kernel,
    mesh=_MESH,
    compiler_params=_SC_PARAMS,
    out_type=jax.ShapeDtypeStruct((BL, 32), jnp.float32),
    scratch_types=[
        pltpu.VMEM((CHUNK,), jnp.int32),
        pltpu.VMEM((CHUNK, 32), jnp.float32),
        pltpu.SemaphoreType.DMA,
    ],
)
def _gather_loc(lt_hbm, p_loc_hbm, out_hbm, idx_v, rows_v, sem):
    wid = lax.axis_index("s") * NC + lax.axis_index("c")

    def chunk(i, carry):
        off = wid * TPW + i * CHUNK
        pltpu.sync_copy(lt_hbm.at[pl.ds(off, CHUNK)], idx_v)
        pltpu.async_copy(p_loc_hbm.at[idx_v], rows_v, sem).wait()
        pltpu.sync_copy(rows_v, out_hbm.at[pl.ds(off, CHUNK)])
        return carry

    lax.fori_loop(0, NCHUNK, chunk, 0)


def kernel(context_tokens, time_tokens, loc_tokens, act_tokens,
           time_table, loc_table, act_table, age_table, gender_table,
           W_time, b_time, W_loc, b_loc, W_act, b_act):
    p_time = _project(time_table, W_time.T, b_time.reshape(1, -1), 1000)
    p_act = _project(act_table, W_act.T, b_act.reshape(1, -1), 1000)
    p_loc = _project(loc_table, W_loc.T, b_loc.reshape(1, -1), 10000)

    comb = jnp.zeros((304, CTX_PAD), jnp.float32)
    comb = comb.at[:300, :2].set(jnp.repeat(gender_table, 100, axis=0))
    comb = comb.at[:300, 2:6].set(jnp.tile(age_table, (3, 1)))

    tt = time_tokens.reshape(-1)
    at_ = act_tokens.reshape(-1)
    lt = loc_tokens.reshape(-1)
    c0 = context_tokens[:, 0]
    c1 = context_tokens[:, 1]

    time_out, act_out, ctx_out = _gather_time_act_ctx(
        tt, at_, c0, c1, p_time, p_act, comb)
    lt, _ = jax.lax.optimization_barrier((lt, time_out))
    loc_out = _gather_loc(lt, p_loc)

    return (
        ctx_out[:, :6],
        time_out.reshape(B, L, 32),
        loc_out.reshape(B, L, 32),
        act_out.reshape(B, L, 32),
    )


# speedup vs baseline: 11.4139x; 2.6658x over previous
import functools

import jax
import jax.numpy as jnp
from jax import lax
from jax.experimental import pallas as pl
from jax.experimental.pallas import tpu as pltpu, tpu_sc as plsc

B = 16384
L = 200
BL = B * L

NC = 2
NS = 16
NW = NC * NS
LANES = 16

BPW = B // NW
CTX_PAD = 16
TV = 1000
LOC_VP = 1024000


def _proj_body(xt_ref, w_ref, b_ref, o_ref):
    y = lax.dot_general(
        xt_ref[...], w_ref[...], (((0,), (1,)), ((), ())),
        preferred_element_type=jnp.float32,
    )
    o_ref[...] = y + b_ref[...]


def _project(xt, w, b_row, blk):
    d_in, v = xt.shape
    return pl.pallas_call(
        _proj_body,
        grid=(v // blk,),
        in_specs=[
            pl.BlockSpec((d_in, blk), lambda i: (0, i)),
            pl.BlockSpec((32, d_in), lambda i: (0, 0)),
            pl.BlockSpec((1, 32), lambda i: (0, 0)),
        ],
        out_specs=pl.BlockSpec((blk, 32), lambda i: (i, 0)),
        out_shape=jax.ShapeDtypeStruct((v, 32), jnp.float32),
    )(xt, w, b_row)


def _retile_body(x_ref, o_ref):
    o_ref[...] = x_ref[...].T


def _retile(z):
    x2 = z.reshape(BL * 32 // 128, 128)
    o2 = pl.pallas_call(
        _retile_body,
        grid=(L // 4, 4),
        in_specs=[pl.BlockSpec((4096, 128), lambda i, j: (i * 4 + j, 0))],
        out_specs=pl.BlockSpec((128, 4096), lambda i, j: (i, j)),
        out_shape=jax.ShapeDtypeStruct((L * 32, B), jnp.float32),
    )(x2)
    return o2.reshape(L, 32, B).transpose(2, 0, 1)


_MESH = plsc.VectorSubcoreMesh(core_axis_name="c", subcore_axis_name="s")
_SC_PARAMS = pltpu.CompilerParams(
    use_tc_tiling_on_sc=False, needs_layout_passes=False)

_SLAB = jax.ShapeDtypeStruct((L // 4, B, 128), jnp.float32)


def _slab_dst(out_hbm, l, b0):
    return out_hbm.at[l // 4, pl.ds(b0, BPW), pl.ds(lax.rem(l, 4) * 32, 32)]


def _ring_streams(streams, b0):
    def fire(l, h):
        for tokT, tab, _out, idx_v, rows_v, gs, _ws in streams:
            pltpu.sync_copy(tokT.at[l, pl.ds(b0, BPW)], idx_v.at[h])
            pltpu.async_copy(tab.at[idx_v.at[h]], rows_v.at[h], gs.at[h])

    def drain_fire_out(l, h):
        for _tokT, tab, out, idx_v, rows_v, gs, ws in streams:
            pltpu.make_async_copy(
                tab.at[idx_v.at[h]], rows_v.at[h], gs.at[h]).wait()
            pltpu.async_copy(rows_v.at[h], _slab_dst(out, l, b0), ws.at[h])

    def wait_out(l, h):
        for _tokT, _tab, out, _idx_v, rows_v, _gs, ws in streams:
            pltpu.make_async_copy(
                rows_v.at[h], _slab_dst(out, l, b0), ws.at[h]).wait()

    for h in range(2):
        fire(h, h)
    for h in range(2):
        drain_fire_out(h, h)

    @pl.loop(2, L, step=2)
    def _rows(i):
        for h in range(2):
            wait_out(i + h - 2, h)
            fire(i + h, h)
        for h in range(2):
            drain_fire_out(i + h, h)

    for h in range(2):
        wait_out(L - 2 + h, h)


def _transpose_chunk16(rows, trows, c):
    giota = lax.iota(jnp.int32, 16)
    for d in range(CTX_PAD):
        dvec = jnp.full((16,), d, jnp.int32)
        for g in range(c // LANES):
            rvec = giota + (g * LANES)
            trows[d, pl.ds(g * LANES, LANES)] = plsc.load_gather(
                rows, [rvec, dvec])


@functools.partial(
    pl.kernel,
    mesh=_MESH,
    compiler_params=_SC_PARAMS,
    out_type=[
        _SLAB,
        jax.ShapeDtypeStruct((CTX_PAD, B), jnp.float32),
    ],
    scratch_types=[
        pltpu.VMEM_SHARED((TV, 32), jnp.float32),
        pltpu.VMEM_SHARED((304, CTX_PAD), jnp.float32),
        pltpu.VMEM((2, BPW), jnp.int32),
        pltpu.VMEM((2, BPW, 32), jnp.float32),
        pltpu.VMEM((BPW,), jnp.int32),
        pltpu.VMEM((BPW,), jnp.int32),
        pltpu.VMEM((BPW,), jnp.int32),
        pltpu.VMEM((BPW, CTX_PAD), jnp.float32),
        pltpu.VMEM((CTX_PAD, BPW), jnp.float32),
        pltpu.SemaphoreType.DMA((2,)),
        pltpu.SemaphoreType.DMA((2,)),
        pltpu.SemaphoreType.DMA,
    ],
)
def _gather_time_ctx(ttT_hbm, c0_hbm, c1_hbm, p_time_hbm, p_ctx_hbm,
                     out_t_hbm, out_c_hbm,
                     ptime_sh, comb_sh, t_idx, t_rows,
                     c0_v, c1_v, cidx_v, crows_v, ctr_v,
                     tg, to, csem):
    cid = lax.axis_index("c")
    sid = lax.axis_index("s")
    wid = sid * NC + cid
    b0 = wid * BPW

    @pl.when(sid == 0)
    def _stage():
        pltpu.sync_copy(p_time_hbm, ptime_sh)
        pltpu.sync_copy(p_ctx_hbm, comb_sh)

    plsc.subcore_barrier()

    pltpu.sync_copy(c0_hbm.at[pl.ds(b0, BPW)], c0_v)
    pltpu.sync_copy(c1_hbm.at[pl.ds(b0, BPW)], c1_v)
    for j in range(BPW // LANES):
        sl = pl.ds(j * LANES, LANES)
        cidx_v[sl] = c0_v[sl] * 100 + c1_v[sl]
    pltpu.async_copy(comb_sh.at[cidx_v], crows_v, csem).wait()
    _transpose_chunk16(crows_v, ctr_v, BPW)
    pltpu.sync_copy(ctr_v, out_c_hbm.at[:, pl.ds(b0, BPW)])

    _ring_streams([(ttT_hbm, ptime_sh, out_t_hbm, t_idx, t_rows, tg, to)], b0)


@functools.partial(
    pl.kernel,
    mesh=_MESH,
    compiler_params=_SC_PARAMS,
    out_type=[_SLAB, _SLAB],
    scratch_types=[
        pltpu.VMEM_SHARED((TV, 32), jnp.float32),
        pltpu.VMEM((2, BPW), jnp.int32),
        pltpu.VMEM((2, BPW, 32), jnp.float32),
        pltpu.VMEM((2, BPW), jnp.int32),
        pltpu.VMEM((2, BPW, 32), jnp.float32),
        pltpu.SemaphoreType.DMA((2,)),
        pltpu.SemaphoreType.DMA((2,)),
        pltpu.SemaphoreType.DMA((2,)),
        pltpu.SemaphoreType.DMA((2,)),
    ],
)
def _gather_act_loc(atT_hbm, ltT_hbm, p_act_hbm, p_loc_hbm,
                    out_a_hbm, out_l_hbm,
                    pact_sh, a_idx, a_rows, l_idx, l_rows,
                    ag, ao, lg, lo):
    cid = lax.axis_index("c")
    sid = lax.axis_index("s")
    wid = sid * NC + cid
    b0 = wid * BPW

    @pl.when(sid == 0)
    def _stage():
        pltpu.sync_copy(p_act_hbm, pact_sh)

    plsc.subcore_barrier()

    _ring_streams(
        [
            (atT_hbm, pact_sh, out_a_hbm, a_idx, a_rows, ag, ao),
            (ltT_hbm, p_loc_hbm, out_l_hbm, l_idx, l_rows, lg, lo),
        ],
        b0,
    )


def kernel(context_tokens, time_tokens, loc_tokens, act_tokens,
           time_table, loc_table, act_table, age_table, gender_table,
           W_time, b_time, W_loc, b_loc, W_act, b_act):
    p_time = _project(time_table.T, W_time, b_time.reshape(1, 32), TV)
    p_act = _project(act_table.T, W_act, b_act.reshape(1, 32), TV)
    ltp = jnp.pad(loc_table.T, ((0, 0), (0, LOC_VP - 1000000)))
    p_loc = _project(ltp, W_loc, b_loc.reshape(1, 32), 12800)

    comb = jnp.zeros((304, CTX_PAD), jnp.float32)
    comb = comb.at[:300, :2].set(jnp.repeat(gender_table, 100, axis=0))
    comb = comb.at[:300, 2:6].set(jnp.tile(age_table, (3, 1)))

    ttT = time_tokens.T
    atT = act_tokens.T
    ltT = loc_tokens.T
    c0 = context_tokens[:, 0]
    c1 = context_tokens[:, 1]

    time_z, ctxT = _gather_time_ctx(ttT, c0, c1, p_time, comb)
    atT, _ = jax.lax.optimization_barrier((atT, time_z))
    act_z, loc_z = _gather_act_loc(atT, ltT, p_act, p_loc)

    return (
        ctxT.T[:, :6],
        _retile(time_z),
        _retile(loc_z),
        _retile(act_z),
    )
